# Initial kernel scaffold; baseline (speedup 1.0000x reference)
import jax
import jax.numpy as jnp
from jax.experimental import pallas as pl


def kernel(x, adj_t, W1, b1, W2, b2, W3, b3, g1, be1, g2, be2):
    raise NotImplementedError("write your pallas kernel here")


# speedup vs baseline: 14.3422x; 14.3422x over previous
import functools

import jax
import jax.numpy as jnp
from jax import lax
from jax.experimental import pallas as pl
from jax.experimental.pallas import tpu as pltpu
from jax.experimental.pallas import tpu_sc as plsc

N = 10000
NP = 10240
D = 128
E = 320000
CHUNK = 128
NCHUNKS = E // CHUNK
NCORE = 2
NSUB = 16
NW = NCORE * NSUB
RPS = NP // NSUB
CPC = NCHUNKS // NCORE


def _deg_body(dst2d, deg_out, idxbuf, hist):
    cid = lax.axis_index("c")
    sid = lax.axis_index("s")
    w = cid * NSUB + sid

    def zstep(i, _):
        hist[pl.ds(i * 16, 16)] = jnp.zeros((16,), jnp.float32)
        return 0
    lax.fori_loop(0, NP // 16, zstep, 0)

    ones16 = jnp.full((16,), 1.0, jnp.float32)
    nt = (NCHUNKS - w + NW - 1) // NW

    def step(t, _):
        j = w + t * NW
        pltpu.sync_copy(dst2d.at[j], idxbuf.at[0])
        for g in range(CHUNK // 16):
            idxv = idxbuf[0, pl.ds(g * 16, 16)]
            plsc.addupdate_scatter(hist, [idxv], ones16)
        return 0
    lax.fori_loop(0, nt, step, 0)

    pltpu.sync_copy(hist, deg_out.at[w])


_deg_kernel = functools.partial(
    pl.kernel,
    out_type=jax.ShapeDtypeStruct((NW, NP), jnp.float32),
    mesh=plsc.VectorSubcoreMesh(core_axis_name="c", subcore_axis_name="s"),
    scratch_types=[
        pltpu.VMEM((1, CHUNK), jnp.int32),
        pltpu.VMEM((NP,), jnp.float32),
    ],
    compiler_params=pltpu.CompilerParams(needs_layout_passes=False),
)(_deg_body)


def _agg_body(hw, src2d, dst2d, p0, p1, idx, msg, zbuf, acc):
    cid = lax.axis_index("c")
    sid = lax.axis_index("s")

    def zfill(i, _):
        for g in range(D // 16):
            zbuf[i, pl.ds(g * 16, 16)] = jnp.zeros((16,), jnp.float32)
        return 0
    lax.fori_loop(0, 64, zfill, 0)

    r0 = sid * RPS
    for c in range(RPS // 64):
        pltpu.sync_copy(zbuf, acc.at[pl.ds(r0 + c * 64, 64)])
    plsc.subcore_barrier()

    nt = (CPC - sid + NSUB - 1) // NSUB

    def step(t, _):
        j = cid * CPC + sid + t * NSUB
        pltpu.sync_copy(src2d.at[j], idx.at[0])
        pltpu.sync_copy(dst2d.at[j], idx.at[1])
        pltpu.sync_copy(hw.at[idx.at[0]], msg)
        pltpu.sync_copy(msg, acc.at[idx.at[1]], add=True)
        return 0
    lax.fori_loop(0, nt, step, 0)

    plsc.subcore_barrier()

    @pl.when(cid == 0)
    def _():
        pltpu.sync_copy(acc.at[pl.ds(r0, RPS)], p0.at[pl.ds(r0, RPS)])

    @pl.when(cid == 1)
    def _():
        pltpu.sync_copy(acc.at[pl.ds(r0, RPS)], p1.at[pl.ds(r0, RPS)])


_agg_kernel = functools.partial(
    pl.kernel,
    out_type=(jax.ShapeDtypeStruct((NP, D), jnp.float32),
              jax.ShapeDtypeStruct((NP, D), jnp.float32)),
    mesh=plsc.VectorSubcoreMesh(core_axis_name="c", subcore_axis_name="s"),
    scratch_types=[
        pltpu.VMEM((2, CHUNK), jnp.int32),
        pltpu.VMEM((CHUNK, D), jnp.float32),
        pltpu.VMEM((64, D), jnp.float32),
        pltpu.VMEM_SHARED((NP, D), jnp.float32),
    ],
)(_agg_body)


def _dinv_from_parts(deg_ref):
    ones = jnp.ones((NW, 1), jnp.float32)
    deg = lax.dot_general(deg_ref[...], ones, (((0,), (0,)), ((), ())),
                          preferred_element_type=jnp.float32)
    return lax.rsqrt(deg[:N, :] + 1.0)


def _tc_l1(x_ref, w_ref, deg_ref, hw_ref):
    dinv = _dinv_from_parts(deg_ref)
    hw = jnp.dot(x_ref[...], w_ref[...], preferred_element_type=jnp.float32)
    hw_ref[:N, :] = hw * dinv
    hw_ref[N:, :] = jnp.zeros((NP - N, D), jnp.float32)


def _tc_mid(p0_ref, p1_ref, hwp_ref, deg_ref, b_ref, g_ref, be_ref, w_ref,
            hw_ref):
    dinv = _dinv_from_parts(deg_ref)
    agg = p0_ref[:N, :] + p1_ref[:N, :] + hwp_ref[:N, :]
    h = agg * dinv + b_ref[...]
    m = jnp.mean(h, axis=0, keepdims=True)
    c = h - m
    v = jnp.mean(c * c, axis=0, keepdims=True)
    h = c * lax.rsqrt(v + 1e-5) * g_ref[...] + be_ref[...]
    h = jnp.maximum(h, 0.0)
    hw = jnp.dot(h, w_ref[...], preferred_element_type=jnp.float32)
    hw_ref[:N, :] = hw * dinv
    hw_ref[N:, :] = jnp.zeros((NP - N, D), jnp.float32)


def _tc_fin(p0_ref, p1_ref, hwp_ref, deg_ref, b_ref, out_ref):
    dinv = _dinv_from_parts(deg_ref)
    agg = p0_ref[:N, :] + p1_ref[:N, :] + hwp_ref[:N, :]
    out_ref[...] = agg * dinv + b_ref[...]


_l1_call = pl.pallas_call(
    _tc_l1,
    out_shape=jax.ShapeDtypeStruct((NP, D), jnp.float32),
)

_mid_call = pl.pallas_call(
    _tc_mid,
    out_shape=jax.ShapeDtypeStruct((NP, D), jnp.float32),
)

_fin_call = pl.pallas_call(
    _tc_fin,
    out_shape=jax.ShapeDtypeStruct((N, D), jnp.float32),
)


@jax.jit
def _run(x, adj_t, W1, b1, W2, b2, W3, b3, g1, be1, g2, be2):
    src2d = adj_t[0].reshape(NCHUNKS, CHUNK)
    dst2d = adj_t[1].reshape(NCHUNKS, CHUNK)
    b1r = b1.reshape(1, D)
    b2r = b2.reshape(1, D)
    b3r = b3.reshape(1, D)
    g1r = g1.reshape(1, D)
    g2r = g2.reshape(1, D)
    be1r = be1.reshape(1, D)
    be2r = be2.reshape(1, D)

    degp = _deg_kernel(dst2d)

    hw = _l1_call(x, W1, degp)
    p0, p1 = _agg_kernel(hw, src2d, dst2d)
    hw2 = _mid_call(p0, p1, hw, degp, b1r, g1r, be1r, W2)
    p0, p1 = _agg_kernel(hw2, src2d, dst2d)
    hw3 = _mid_call(p0, p1, hw2, degp, b2r, g2r, be2r, W3)
    p0, p1 = _agg_kernel(hw3, src2d, dst2d)
    return _fin_call(p0, p1, hw3, degp, b3r)


def kernel(x, adj_t, W1, b1, W2, b2, W3, b3, g1, be1, g2, be2):
    return _run(x, adj_t, W1, b1, W2, b2, W3, b3, g1, be1, g2, be2)
